# baseline (speedup 1.0000x reference)
import jax
import jax.numpy as jnp
from jax.experimental import pallas as pl
from jax.experimental.pallas import tpu as pltpu

DEPTH_ = 100000
BATCH_ = 1024

BLOCK_R = 256
BLOCK_C = 2048


def _onehot_block(idx_ref, out_ref):
    j = pl.program_id(1)
    idx = idx_ref[...]
    col = jax.lax.broadcasted_iota(jnp.int32, (BLOCK_R, BLOCK_C), 1)
    col = col + j * BLOCK_C
    out_ref[...] = (col == idx).astype(jnp.float32)


def kernel(input):
    idx = input.astype(jnp.int32)
    grid = (BATCH_ // BLOCK_R, pl.cdiv(DEPTH_, BLOCK_C))
    out = pl.pallas_call(
        _onehot_block,
        grid=grid,
        in_specs=[pl.BlockSpec((BLOCK_R, 1), lambda i, j: (i, 0))],
        out_specs=pl.BlockSpec((BLOCK_R, BLOCK_C), lambda i, j: (i, j)),
        out_shape=jax.ShapeDtypeStruct((BATCH_, DEPTH_), jnp.float32),
    )(idx)
    return out


# speedup vs baseline: 1.0676x; 1.0676x over previous
import jax
import jax.numpy as jnp
from jax.experimental import pallas as pl
from jax.experimental.pallas import tpu as pltpu

DEPTH_ = 100000
BATCH_ = 1024

BLOCK_R = 16


def _onehot_block(idx_ref, out_ref):
    idx = idx_ref[...]
    col = jax.lax.broadcasted_iota(jnp.int32, (BLOCK_R, DEPTH_), 1)
    out_ref[...] = (col == idx).astype(jnp.float32)


def kernel(input):
    idx = input.astype(jnp.int32)
    grid = (BATCH_ // BLOCK_R,)
    out = pl.pallas_call(
        _onehot_block,
        grid=grid,
        in_specs=[pl.BlockSpec((BLOCK_R, 1), lambda i: (i, 0))],
        out_specs=pl.BlockSpec((BLOCK_R, DEPTH_), lambda i: (i, 0)),
        out_shape=jax.ShapeDtypeStruct((BATCH_, DEPTH_), jnp.float32),
    )(idx)
    return out
